# baseline (device time: 833017 ns/iter reference)
import jax
import jax.numpy as jnp
from jax import lax
from jax.experimental import pallas as pl
from jax.experimental.pallas import tpu as pltpu

N_DEV = 4
SCALE = 0.08838834764831843
BLK = 64
NBLK = 32
GP = 11 * BLK
NG = 3
P3 = NG * GP


def kernel(x, Wq, K_ext, V_ext, Wo):
    _, S, D = x.shape
    H, Dh = K_ext.shape[2], K_ext.shape[3]

    my_out = lax.axis_index("i")

    i2 = jnp.arange(P3, dtype=jnp.int32)
    g_i = i2 // GP
    kblk = (i2 % GP) // BLK
    o_i = i2 % BLK
    beta = (g_i - 2 * my_out) % 3
    cnt = (34 - beta) // 3
    ksafe = jnp.minimum(kblk, cnt - 1)
    src = (beta + 3 * ksafe) * BLK + o_i

    q = (x[0] @ Wq) * SCALE
    q = jnp.take(q, src, axis=0)
    q = q.reshape(P3, H, Dh).transpose(1, 0, 2).astype(jnp.bfloat16)
    k = jnp.take(K_ext[0], src, axis=0).transpose(1, 0, 2).astype(jnp.bfloat16)
    v = jnp.take(V_ext[0], src, axis=0).transpose(1, 0, 2).astype(jnp.bfloat16)
    wo = Wo.astype(jnp.bfloat16)

    def body(q_ref, k_ref, v_ref, wo_ref, out_ref, kg, vg,
             acc, kc, vc, m_sc, l_sc, copy_sems, chunk_sems,
             ksend, krecv, vsend, vrecv):
        my = lax.axis_index("i")

        ck = pltpu.make_async_copy(k_ref, kg.at[my], copy_sems.at[0])
        cv = pltpu.make_async_copy(v_ref, vg.at[my], copy_sems.at[1])
        ck.start()
        cv.start()

        bsem = pltpu.get_barrier_semaphore()
        for off in (1, 2, 3):
            pl.semaphore_signal(bsem, inc=1,
                                device_id=(lax.rem(my + off, N_DEV),),
                                device_id_type=pl.DeviceIdType.MESH)
        pl.semaphore_wait(bsem, 3)

        ck.wait()
        cv.wait()

        sends = []
        for off in (3, 1, 2):
            p = lax.rem(my + off, N_DEV)
            slot = 3 - off
            rk = pltpu.make_async_remote_copy(
                src_ref=kg.at[my], dst_ref=kg.at[my],
                send_sem=ksend.at[slot], recv_sem=krecv.at[slot],
                device_id=(p,), device_id_type=pl.DeviceIdType.MESH)
            rv = pltpu.make_async_remote_copy(
                src_ref=vg.at[my], dst_ref=vg.at[my],
                send_sem=vsend.at[slot], recv_sem=vrecv.at[slot],
                device_id=(p,), device_id_type=pl.DeviceIdType.MESH)
            rk.start()
            rv.start()
            sends += [rk, rv]

        acc[...] = jnp.zeros_like(acc)
        m_sc[...] = jnp.full_like(m_sc, -1e30)
        l_sc[...] = jnp.zeros_like(l_sc)

        def flash_update(hh, g, s, v_op):
            mh = jnp.reshape(m_sc[hh, g, :], (GP, 1))
            lh = jnp.reshape(l_sc[hh, g, :], (GP, 1))
            cmax = jnp.max(s, axis=1, keepdims=True)
            mn = jnp.maximum(mh, cmax)
            alpha = jnp.exp(mh - mn)
            p = jnp.exp(s - mn)
            ln = alpha * lh + jnp.sum(p, axis=1, keepdims=True)
            pv = lax.dot_general(
                p.astype(jnp.bfloat16), v_op, (((1,), (0,)), ((), ())),
                preferred_element_type=jnp.float32)
            prev = acc[hh, g * GP:(g + 1) * GP, :]
            acc[hh, g * GP:(g + 1) * GP, :] = alpha * prev + pv
            m_sc[hh, g, :] = jnp.reshape(mn, (GP,))
            l_sc[hh, g, :] = jnp.reshape(ln, (GP,))

        for off in (0, 1, 3, 2):
            c = lax.rem(my + off, N_DEV)
            if off != 0:
                slot = off - 1
                wk = pltpu.make_async_remote_copy(
                    src_ref=kg.at[c], dst_ref=kg.at[c],
                    send_sem=ksend.at[slot], recv_sem=krecv.at[slot],
                    device_id=(my,), device_id_type=pl.DeviceIdType.MESH)
                wv = pltpu.make_async_remote_copy(
                    src_ref=vg.at[c], dst_ref=vg.at[c],
                    send_sem=vsend.at[slot], recv_sem=vrecv.at[slot],
                    device_id=(my,), device_id_type=pl.DeviceIdType.MESH)
                wk.wait_recv()
                wv.wait_recv()

            fk = pltpu.make_async_copy(kg.at[c], kc, chunk_sems.at[0])
            fv = pltpu.make_async_copy(vg.at[c], vc, chunk_sems.at[1])
            fk.start()
            fv.start()
            fk.wait()
            fv.wait()

            for g in range(NG):
                r = (3 - g) % 3
                beta_k = lax.rem(r - 2 * c + 12, 3)
                kcnt = (34 - beta_k) // 3
                col_ok = lax.broadcasted_iota(jnp.int32, (1, GP), 1) < kcnt * BLK
                bias = jnp.where(col_ok, 0.0, -1e9).astype(jnp.float32)

                def reg_head(hh, _, g=g, r=r, bias=bias):
                    qh = q_ref[hh, g * GP:(g + 1) * GP, :]
                    s = lax.dot_general(
                        qh, kc[hh, r * GP:(r + 1) * GP, :],
                        (((1,), (1,)), ((), ())),
                        preferred_element_type=jnp.float32) + bias
                    flash_update(hh, g, s, vc[hh, r * GP:(r + 1) * GP, :])
                    return 0

                lax.fori_loop(0, H, reg_head, 0)

            @pl.when(c == 0)
            def _kb0():
                for g in (1, 2):
                    for hh in range(H):
                        qh = q_ref[hh, g * GP:(g + 1) * GP, :]
                        s = lax.dot_general(
                            qh, kc[hh, 0:BLK, :], (((1,), (1,)), ((), ())),
                            preferred_element_type=jnp.float32)
                        flash_update(hh, g, s, vc[hh, 0:BLK, :])

            if off == 0:
                rio = lax.broadcasted_iota(jnp.int32, (GP, GP), 0) // BLK
                cio = lax.broadcasted_iota(jnp.int32, (GP, GP), 1) // BLK
                dkeep = rio == cio
                dbias = jnp.where(dkeep, 0.0, -1e9).astype(jnp.float32)
                for g in (1, 2):
                    for hh in range(H):
                        qh = q_ref[hh, g * GP:(g + 1) * GP, :]
                        s = lax.dot_general(
                            qh, kc[hh, g * GP:(g + 1) * GP, :],
                            (((1,), (1,)), ((), ())),
                            preferred_element_type=jnp.float32) + dbias
                        flash_update(hh, g, s, vc[hh, g * GP:(g + 1) * GP, :])

        for r_ in sends:
            r_.wait_send()

        for g in range(NG):
            for hh in range(H):
                lh = jnp.reshape(l_sc[hh, g, :], (GP, 1))
                out_ref[g * GP:(g + 1) * GP, hh * Dh:(hh + 1) * Dh] = (
                    acc[hh, g * GP:(g + 1) * GP, :] / lh)

        ctx = out_ref[...].astype(jnp.bfloat16)
        out_ref[...] = lax.dot_general(
            ctx, wo_ref[...], (((1,), (0,)), ((), ())),
            preferred_element_type=jnp.float32)

    out, _, _ = pl.pallas_call(
        body,
        out_shape=(
            jax.ShapeDtypeStruct((P3, D), jnp.float32),
            jax.ShapeDtypeStruct((N_DEV, H, P3, Dh), jnp.bfloat16),
            jax.ShapeDtypeStruct((N_DEV, H, P3, Dh), jnp.bfloat16),
        ),
        in_specs=[
            pl.BlockSpec(memory_space=pltpu.VMEM),
            pl.BlockSpec(memory_space=pltpu.HBM),
            pl.BlockSpec(memory_space=pltpu.HBM),
            pl.BlockSpec(memory_space=pltpu.VMEM),
        ],
        out_specs=(
            pl.BlockSpec(memory_space=pltpu.VMEM),
            pl.BlockSpec(memory_space=pltpu.HBM),
            pl.BlockSpec(memory_space=pltpu.HBM),
        ),
        scratch_shapes=[
            pltpu.VMEM((H, P3, Dh), jnp.float32),
            pltpu.VMEM((H, P3, Dh), jnp.bfloat16),
            pltpu.VMEM((H, P3, Dh), jnp.bfloat16),
            pltpu.VMEM((H, NG, GP), jnp.float32),
            pltpu.VMEM((H, NG, GP), jnp.float32),
            pltpu.SemaphoreType.DMA((2,)),
            pltpu.SemaphoreType.DMA((2,)),
            pltpu.SemaphoreType.DMA((3,)),
            pltpu.SemaphoreType.DMA((3,)),
            pltpu.SemaphoreType.DMA((3,)),
            pltpu.SemaphoreType.DMA((3,)),
        ],
        compiler_params=pltpu.CompilerParams(
            collective_id=0, vmem_limit_bytes=56 * 1024 * 1024),
    )(q, k, v, wo)

    n = jnp.arange(S, dtype=jnp.int32)
    b = n // BLK
    o = n % BLK
    gq = (2 * my_out + b) % 3
    beta_n = (gq - 2 * my_out) % 3
    kpos = (b - beta_n) // 3
    idx = gq * GP + kpos * BLK + o
    return jnp.take(out, idx, axis=0)[None]


# device time: 729382 ns/iter; 1.1421x vs baseline; 1.1421x over previous
import jax
import jax.numpy as jnp
from jax import lax
from jax.experimental import pallas as pl
from jax.experimental.pallas import tpu as pltpu

N_DEV = 4
SCALE = 0.08838834764831843
BLK = 64
NBLK = 32
GP = 11 * BLK
NG = 3
P3 = NG * GP


def kernel(x, Wq, K_ext, V_ext, Wo):
    _, S, D = x.shape
    H, Dh = K_ext.shape[2], K_ext.shape[3]

    my_out = lax.axis_index("i")
    t_res = (2 * my_out) % 3

    def pack_rows(X):
        Xb = X.reshape(NBLK, BLK, -1)
        grp = []
        for s_ in range(3):
            Gb = Xb[s_::3]
            if Gb.shape[0] < 11:
                Gb = jnp.concatenate([Gb, Gb[-1:]], axis=0)
            grp.append(Gb)

        def variant(tt):
            return jnp.concatenate(
                [grp[(0 - tt) % 3], grp[(1 - tt) % 3], grp[(2 - tt) % 3]],
                axis=0)

        Y = lax.switch(t_res, [lambda tt=tt: variant(tt) for tt in range(3)])
        return Y.reshape(P3, *X.shape[1:])

    def unpack_rows(Y):
        Gb = Y.reshape(3, 11, BLK, -1)

        def variant(tt):
            z = jnp.stack(
                [Gb[(tt + 0) % 3], Gb[(tt + 1) % 3], Gb[(tt + 2) % 3]],
                axis=1)
            return z.reshape(33, BLK, -1)[:NBLK]

        Z = lax.switch(t_res, [lambda tt=tt: variant(tt) for tt in range(3)])
        return Z.reshape(S, *Y.shape[1:])

    q = ((x[0] @ Wq) * SCALE).astype(jnp.bfloat16)
    q = pack_rows(q).reshape(P3, H, Dh).transpose(1, 0, 2)
    k = pack_rows(K_ext[0].astype(jnp.bfloat16).reshape(S, H * Dh))
    k = k.reshape(P3, H, Dh).transpose(1, 0, 2)
    v = pack_rows(V_ext[0].astype(jnp.bfloat16).reshape(S, H * Dh))
    v = v.reshape(P3, H, Dh).transpose(1, 0, 2)
    wo = Wo.astype(jnp.bfloat16)

    def body(q_ref, k_ref, v_ref, wo_ref, out_ref, kg, vg,
             acc, kc, vc, m_sc, l_sc, copy_sems, chunk_sems,
             ksend, krecv, vsend, vrecv):
        my = lax.axis_index("i")

        ck = pltpu.make_async_copy(k_ref, kg.at[my], copy_sems.at[0])
        cv = pltpu.make_async_copy(v_ref, vg.at[my], copy_sems.at[1])
        ck.start()
        cv.start()

        bsem = pltpu.get_barrier_semaphore()
        for off in (1, 2, 3):
            pl.semaphore_signal(bsem, inc=1,
                                device_id=(lax.rem(my + off, N_DEV),),
                                device_id_type=pl.DeviceIdType.MESH)
        pl.semaphore_wait(bsem, 3)

        ck.wait()
        cv.wait()

        sends = []
        for off in (3, 1, 2):
            p = lax.rem(my + off, N_DEV)
            slot = 3 - off
            rk = pltpu.make_async_remote_copy(
                src_ref=kg.at[my], dst_ref=kg.at[my],
                send_sem=ksend.at[slot], recv_sem=krecv.at[slot],
                device_id=(p,), device_id_type=pl.DeviceIdType.MESH)
            rv = pltpu.make_async_remote_copy(
                src_ref=vg.at[my], dst_ref=vg.at[my],
                send_sem=vsend.at[slot], recv_sem=vrecv.at[slot],
                device_id=(p,), device_id_type=pl.DeviceIdType.MESH)
            rk.start()
            rv.start()
            sends += [rk, rv]

        acc[...] = jnp.zeros_like(acc)
        m_sc[...] = jnp.full_like(m_sc, -1e30)
        l_sc[...] = jnp.zeros_like(l_sc)

        def flash_update(hh, g, s, v_op):
            mh = jnp.reshape(m_sc[hh, g, :], (GP, 1))
            lh = jnp.reshape(l_sc[hh, g, :], (GP, 1))
            cmax = jnp.max(s, axis=1, keepdims=True)
            mn = jnp.maximum(mh, cmax)
            alpha = jnp.exp(mh - mn)
            p = jnp.exp(s - mn)
            ln = alpha * lh + jnp.sum(p, axis=1, keepdims=True)
            pv = lax.dot_general(
                p.astype(jnp.bfloat16), v_op, (((1,), (0,)), ((), ())),
                preferred_element_type=jnp.float32)
            prev = acc[hh, g * GP:(g + 1) * GP, :]
            acc[hh, g * GP:(g + 1) * GP, :] = alpha * prev + pv
            m_sc[hh, g, :] = jnp.reshape(mn, (GP,))
            l_sc[hh, g, :] = jnp.reshape(ln, (GP,))

        for off in (0, 1, 3, 2):
            c = lax.rem(my + off, N_DEV)
            if off != 0:
                slot = off - 1
                wk = pltpu.make_async_remote_copy(
                    src_ref=kg.at[c], dst_ref=kg.at[c],
                    send_sem=ksend.at[slot], recv_sem=krecv.at[slot],
                    device_id=(my,), device_id_type=pl.DeviceIdType.MESH)
                wv = pltpu.make_async_remote_copy(
                    src_ref=vg.at[c], dst_ref=vg.at[c],
                    send_sem=vsend.at[slot], recv_sem=vrecv.at[slot],
                    device_id=(my,), device_id_type=pl.DeviceIdType.MESH)
                wk.wait_recv()
                wv.wait_recv()

            fk = pltpu.make_async_copy(kg.at[c], kc, chunk_sems.at[0])
            fv = pltpu.make_async_copy(vg.at[c], vc, chunk_sems.at[1])
            fk.start()
            fv.start()
            fk.wait()
            fv.wait()

            for g in range(NG):
                r = (3 - g) % 3
                beta_k = lax.rem(r - 2 * c + 12, 3)
                kcnt = (34 - beta_k) // 3
                col_ok = lax.broadcasted_iota(jnp.int32, (1, GP), 1) < kcnt * BLK
                bias = jnp.where(col_ok, 0.0, -1e9).astype(jnp.float32)

                def reg_head(hh, _, g=g, r=r, bias=bias):
                    qh = q_ref[hh, g * GP:(g + 1) * GP, :]
                    s = lax.dot_general(
                        qh, kc[hh, r * GP:(r + 1) * GP, :],
                        (((1,), (1,)), ((), ())),
                        preferred_element_type=jnp.float32) + bias
                    flash_update(hh, g, s, vc[hh, r * GP:(r + 1) * GP, :])
                    return 0

                lax.fori_loop(0, H, reg_head, 0)

            @pl.when(c == 0)
            def _kb0():
                for g in (1, 2):
                    for hh in range(H):
                        qh = q_ref[hh, g * GP:(g + 1) * GP, :]
                        s = lax.dot_general(
                            qh, kc[hh, 0:BLK, :], (((1,), (1,)), ((), ())),
                            preferred_element_type=jnp.float32)
                        flash_update(hh, g, s, vc[hh, 0:BLK, :])

            if off == 0:
                rio = lax.broadcasted_iota(jnp.int32, (GP, GP), 0) // BLK
                cio = lax.broadcasted_iota(jnp.int32, (GP, GP), 1) // BLK
                dkeep = rio == cio
                dbias = jnp.where(dkeep, 0.0, -1e9).astype(jnp.float32)
                for g in (1, 2):
                    for hh in range(H):
                        qh = q_ref[hh, g * GP:(g + 1) * GP, :]
                        s = lax.dot_general(
                            qh, kc[hh, g * GP:(g + 1) * GP, :],
                            (((1,), (1,)), ((), ())),
                            preferred_element_type=jnp.float32) + dbias
                        flash_update(hh, g, s, vc[hh, g * GP:(g + 1) * GP, :])

        for r_ in sends:
            r_.wait_send()

        for g in range(NG):
            for hh in range(H):
                lh = jnp.reshape(l_sc[hh, g, :], (GP, 1))
                out_ref[g * GP:(g + 1) * GP, hh * Dh:(hh + 1) * Dh] = (
                    acc[hh, g * GP:(g + 1) * GP, :] / lh)

        ctx = out_ref[...].astype(jnp.bfloat16)
        out_ref[...] = lax.dot_general(
            ctx, wo_ref[...], (((1,), (0,)), ((), ())),
            preferred_element_type=jnp.float32)

    out, _, _ = pl.pallas_call(
        body,
        out_shape=(
            jax.ShapeDtypeStruct((P3, D), jnp.float32),
            jax.ShapeDtypeStruct((N_DEV, H, P3, Dh), jnp.bfloat16),
            jax.ShapeDtypeStruct((N_DEV, H, P3, Dh), jnp.bfloat16),
        ),
        in_specs=[
            pl.BlockSpec(memory_space=pltpu.VMEM),
            pl.BlockSpec(memory_space=pltpu.HBM),
            pl.BlockSpec(memory_space=pltpu.HBM),
            pl.BlockSpec(memory_space=pltpu.VMEM),
        ],
        out_specs=(
            pl.BlockSpec(memory_space=pltpu.VMEM),
            pl.BlockSpec(memory_space=pltpu.HBM),
            pl.BlockSpec(memory_space=pltpu.HBM),
        ),
        scratch_shapes=[
            pltpu.VMEM((H, P3, Dh), jnp.float32),
            pltpu.VMEM((H, P3, Dh), jnp.bfloat16),
            pltpu.VMEM((H, P3, Dh), jnp.bfloat16),
            pltpu.VMEM((H, NG, GP), jnp.float32),
            pltpu.VMEM((H, NG, GP), jnp.float32),
            pltpu.SemaphoreType.DMA((2,)),
            pltpu.SemaphoreType.DMA((2,)),
            pltpu.SemaphoreType.DMA((3,)),
            pltpu.SemaphoreType.DMA((3,)),
            pltpu.SemaphoreType.DMA((3,)),
            pltpu.SemaphoreType.DMA((3,)),
        ],
        compiler_params=pltpu.CompilerParams(
            collective_id=0, vmem_limit_bytes=56 * 1024 * 1024),
    )(q, k, v, wo)

    return unpack_rows(out)[None]


# device time: 681202 ns/iter; 1.2229x vs baseline; 1.0707x over previous
import jax
import jax.numpy as jnp
from jax import lax
from jax.experimental import pallas as pl
from jax.experimental.pallas import tpu as pltpu

N_DEV = 4
SCALE = 0.08838834764831843
BLK = 64
NBLK = 32
GP = 11 * BLK
NG = 3
P3 = NG * GP


def kernel(x, Wq, K_ext, V_ext, Wo):
    _, S, D = x.shape
    H, Dh = K_ext.shape[2], K_ext.shape[3]

    my_out = lax.axis_index("i")
    t_res = (2 * my_out) % 3

    def pack_rows(X):
        Xb = X.reshape(NBLK, BLK, -1)
        grp = []
        for s_ in range(3):
            Gb = Xb[s_::3]
            if Gb.shape[0] < 11:
                Gb = jnp.concatenate([Gb, Gb[-1:]], axis=0)
            grp.append(Gb)

        def variant(tt):
            return jnp.concatenate(
                [grp[(0 - tt) % 3], grp[(1 - tt) % 3], grp[(2 - tt) % 3]],
                axis=0)

        Y = lax.switch(t_res, [lambda tt=tt: variant(tt) for tt in range(3)])
        return Y.reshape(P3, *X.shape[1:])

    def unpack_rows(Y):
        Gb = Y.reshape(3, 11, BLK, -1)

        def variant(tt):
            z = jnp.stack(
                [Gb[(tt + 0) % 3], Gb[(tt + 1) % 3], Gb[(tt + 2) % 3]],
                axis=1)
            return z.reshape(33, BLK, -1)[:NBLK]

        Z = lax.switch(t_res, [lambda tt=tt: variant(tt) for tt in range(3)])
        return Z.reshape(S, *Y.shape[1:])

    q = ((x[0] @ Wq) * SCALE).astype(jnp.bfloat16)
    q = pack_rows(q).reshape(P3, H, Dh).transpose(1, 0, 2)
    k = pack_rows(K_ext[0].astype(jnp.bfloat16).reshape(S, H * Dh))
    k = k.reshape(P3, H, Dh).transpose(1, 0, 2)
    v = pack_rows(V_ext[0].astype(jnp.bfloat16).reshape(S, H * Dh))
    v = v.reshape(P3, H, Dh).transpose(1, 0, 2)
    wo = Wo.astype(jnp.bfloat16)

    def body(q_ref, k_ref, v_ref, wo_ref, out_ref, kg, vg,
             acc, kc, vc, m_sc, l_sc, copy_sems, chunk_sems,
             ksend, krecv, vsend, vrecv):
        my = lax.axis_index("i")

        ck = pltpu.make_async_copy(k_ref, kg.at[my], copy_sems.at[0])
        cv = pltpu.make_async_copy(v_ref, vg.at[my], copy_sems.at[1])
        ck.start()
        cv.start()

        bsem = pltpu.get_barrier_semaphore()
        for off in (1, 2, 3):
            pl.semaphore_signal(bsem, inc=1,
                                device_id=(lax.rem(my + off, N_DEV),),
                                device_id_type=pl.DeviceIdType.MESH)
        pl.semaphore_wait(bsem, 3)

        ck.wait()
        cv.wait()

        sends = []
        for off in (3, 1, 2):
            p = lax.rem(my + off, N_DEV)
            slot = 3 - off
            rk = pltpu.make_async_remote_copy(
                src_ref=kg.at[my], dst_ref=kg.at[my],
                send_sem=ksend.at[slot], recv_sem=krecv.at[slot],
                device_id=(p,), device_id_type=pl.DeviceIdType.MESH)
            rv = pltpu.make_async_remote_copy(
                src_ref=vg.at[my], dst_ref=vg.at[my],
                send_sem=vsend.at[slot], recv_sem=vrecv.at[slot],
                device_id=(p,), device_id_type=pl.DeviceIdType.MESH)
            rk.start()
            rv.start()
            sends += [rk, rv]

        acc[...] = jnp.zeros_like(acc)
        m_sc[...] = jnp.full_like(m_sc, -1e30)
        l_sc[...] = jnp.zeros_like(l_sc)

        def flash_update(hh, g, s, v_op):
            mh = m_sc[hh, g].astype(jnp.float32)
            lh = l_sc[hh, g].astype(jnp.float32)
            cmax = jnp.max(s, axis=1, keepdims=True)
            mn = jnp.maximum(mh, cmax)
            alpha = jnp.exp(mh - mn)
            p = jnp.exp(s - mn[:, 0:1])
            ln = alpha * lh + jnp.sum(p, axis=1, keepdims=True)
            pv = lax.dot_general(
                p.astype(jnp.bfloat16), v_op, (((1,), (0,)), ((), ())),
                preferred_element_type=jnp.float32)
            prev = acc[hh, g * GP:(g + 1) * GP, :]
            acc[hh, g * GP:(g + 1) * GP, :] = alpha[:, 0:1] * prev + pv
            m_sc[hh, g] = mn.astype(jnp.bfloat16)
            l_sc[hh, g] = ln.astype(jnp.bfloat16)

        for off in (0, 1, 3, 2):
            c = lax.rem(my + off, N_DEV)
            if off != 0:
                slot = off - 1
                wk = pltpu.make_async_remote_copy(
                    src_ref=kg.at[c], dst_ref=kg.at[c],
                    send_sem=ksend.at[slot], recv_sem=krecv.at[slot],
                    device_id=(my,), device_id_type=pl.DeviceIdType.MESH)
                wv = pltpu.make_async_remote_copy(
                    src_ref=vg.at[c], dst_ref=vg.at[c],
                    send_sem=vsend.at[slot], recv_sem=vrecv.at[slot],
                    device_id=(my,), device_id_type=pl.DeviceIdType.MESH)
                wk.wait_recv()
                wv.wait_recv()

            fk = pltpu.make_async_copy(kg.at[c], kc, chunk_sems.at[0])
            fv = pltpu.make_async_copy(vg.at[c], vc, chunk_sems.at[1])
            fk.start()
            fv.start()
            fk.wait()
            fv.wait()

            for g in range(NG):
                r = (3 - g) % 3
                beta_k = lax.rem(r - 2 * c + 12, 3)
                kcnt = (34 - beta_k) // 3
                col_ok = lax.broadcasted_iota(jnp.int32, (1, GP), 1) < kcnt * BLK
                bias = jnp.where(col_ok, 0.0, -1e9).astype(jnp.float32)

                def reg_head(hh, _, g=g, r=r, bias=bias):
                    qh = q_ref[hh, g * GP:(g + 1) * GP, :]
                    s = lax.dot_general(
                        qh, kc[hh, r * GP:(r + 1) * GP, :],
                        (((1,), (1,)), ((), ())),
                        preferred_element_type=jnp.float32) + bias
                    flash_update(hh, g, s, vc[hh, r * GP:(r + 1) * GP, :])
                    return 0

                lax.fori_loop(0, H, reg_head, 0)

            @pl.when(c == 0)
            def _kb0():
                for g in (1, 2):
                    for hh in range(H):
                        qh = q_ref[hh, g * GP:(g + 1) * GP, :]
                        s = lax.dot_general(
                            qh, kc[hh, 0:BLK, :], (((1,), (1,)), ((), ())),
                            preferred_element_type=jnp.float32)
                        flash_update(hh, g, s, vc[hh, 0:BLK, :])

            if off == 0:
                rio = lax.broadcasted_iota(jnp.int32, (GP, GP), 0) // BLK
                cio = lax.broadcasted_iota(jnp.int32, (GP, GP), 1) // BLK
                dkeep = rio == cio
                dbias = jnp.where(dkeep, 0.0, -1e9).astype(jnp.float32)
                for g in (1, 2):
                    for hh in range(H):
                        qh = q_ref[hh, g * GP:(g + 1) * GP, :]
                        s = lax.dot_general(
                            qh, kc[hh, g * GP:(g + 1) * GP, :],
                            (((1,), (1,)), ((), ())),
                            preferred_element_type=jnp.float32) + dbias
                        flash_update(hh, g, s, vc[hh, g * GP:(g + 1) * GP, :])

        for r_ in sends:
            r_.wait_send()

        for g in range(NG):
            for hh in range(H):
                lh = l_sc[hh, g][:, 0:1].astype(jnp.float32)
                out_ref[g * GP:(g + 1) * GP, hh * Dh:(hh + 1) * Dh] = (
                    acc[hh, g * GP:(g + 1) * GP, :] / lh)

        ctx = out_ref[...].astype(jnp.bfloat16)
        out_ref[...] = lax.dot_general(
            ctx, wo_ref[...], (((1,), (0,)), ((), ())),
            preferred_element_type=jnp.float32)

    out, _, _ = pl.pallas_call(
        body,
        out_shape=(
            jax.ShapeDtypeStruct((P3, D), jnp.float32),
            jax.ShapeDtypeStruct((N_DEV, H, P3, Dh), jnp.bfloat16),
            jax.ShapeDtypeStruct((N_DEV, H, P3, Dh), jnp.bfloat16),
        ),
        in_specs=[
            pl.BlockSpec(memory_space=pltpu.VMEM),
            pl.BlockSpec(memory_space=pltpu.HBM),
            pl.BlockSpec(memory_space=pltpu.HBM),
            pl.BlockSpec(memory_space=pltpu.VMEM),
        ],
        out_specs=(
            pl.BlockSpec(memory_space=pltpu.VMEM),
            pl.BlockSpec(memory_space=pltpu.HBM),
            pl.BlockSpec(memory_space=pltpu.HBM),
        ),
        scratch_shapes=[
            pltpu.VMEM((H, P3, Dh), jnp.float32),
            pltpu.VMEM((H, P3, Dh), jnp.bfloat16),
            pltpu.VMEM((H, P3, Dh), jnp.bfloat16),
            pltpu.VMEM((H, NG, GP, 128), jnp.bfloat16),
            pltpu.VMEM((H, NG, GP, 128), jnp.bfloat16),
            pltpu.SemaphoreType.DMA((2,)),
            pltpu.SemaphoreType.DMA((2,)),
            pltpu.SemaphoreType.DMA((3,)),
            pltpu.SemaphoreType.DMA((3,)),
            pltpu.SemaphoreType.DMA((3,)),
            pltpu.SemaphoreType.DMA((3,)),
        ],
        compiler_params=pltpu.CompilerParams(
            collective_id=0, vmem_limit_bytes=56 * 1024 * 1024),
    )(q, k, v, wo)

    return unpack_rows(out)[None]
